# Initial kernel scaffold; baseline (speedup 1.0000x reference)
import jax
import jax.numpy as jnp
from jax.experimental import pallas as pl


def kernel(input_logits, cu_seqlens_q, relative_idx, batch_offsets, cu_filtered, temperatures, num_transfer, top_k, top_p):
    raise NotImplementedError("write your pallas kernel here")


# speedup vs baseline: 36.5154x; 36.5154x over previous
import functools

import jax
import jax.numpy as jnp
from jax.experimental import pallas as pl
from jax.experimental.pallas import tpu as pltpu

_NEG = float('-inf')
_NUM_EXTRACT = 63


def _sampler_kernel(rows_ref, temps_ref, ks_ref, ps_ref, logits_ref,
                    score_ref, samp_ref, *, width):
  t = pl.program_id(0)
  temp = temps_ref[t]
  k = ks_ref[t]
  p = ps_ref[t]

  z = logits_ref[0]

  sub = jax.lax.broadcasted_iota(jnp.int32, z.shape, 0)
  lanec = jax.lax.broadcasted_iota(jnp.int32, z.shape, 1)
  colidx = sub * width + lanec
  m0 = jnp.max(z)
  amax = jnp.min(jnp.where(z == m0, colidx, jnp.int32(2**30)))

  lane = jax.lax.broadcasted_iota(jnp.int32, (1, 128), 1)

  def body(i, carry):
    m, vals, cnts = carry
    cnt = jnp.sum(jnp.where(z == m, 1.0, 0.0))
    vals = jnp.where(lane == i, m, vals)
    cnts = jnp.where(lane == i, cnt, cnts)
    m2 = jnp.max(jnp.where(z < m, z, _NEG))
    return m2, vals, cnts

  vals0 = jnp.full((1, 128), _NEG, dtype=jnp.float32)
  cnts0 = jnp.zeros((1, 128), dtype=jnp.float32)
  _, vals, cnts = jax.lax.fori_loop(0, _NUM_EXTRACT, body, (m0, vals0, cnts0))

  e = jnp.exp((vals - m0) / temp)
  ce = cnts * e
  tri = (jax.lax.broadcasted_iota(jnp.int32, (128, 128), 0)
         <= jax.lax.broadcasted_iota(jnp.int32, (128, 128), 1)).astype(
             jnp.float32)
  cumcnt = jax.lax.dot(cnts, tri, precision=jax.lax.Precision.HIGHEST)
  cumce = jax.lax.dot(ce, tri, precision=jax.lax.Precision.HIGHEST)

  kf = jnp.clip(k, 1, _NUM_EXTRACT).astype(jnp.float32)
  jstar = jnp.min(jnp.where(cumcnt >= kf, lane, jnp.int32(999)))
  tau = jnp.max(jnp.where(lane == jstar, vals, _NEG))
  pmask = vals >= tau

  e_tot = jnp.sum(jnp.where(pmask, ce, 0.0))
  thr = p * e_tot
  c_before = cumce - ce
  epos = e > 0.0
  kept = jnp.floor((thr - c_before) / jnp.where(epos, e, 1.0)) + 1.0
  kept = jnp.clip(kept, 0.0, cnts)
  kept = jnp.where(pmask & epos, kept, 0.0)
  s_kept = jnp.sum(kept * e)
  score = 1.0 / s_kept

  score_ref[...] = jnp.broadcast_to(score, (1, 1, 128))
  samp_ref[...] = jnp.broadcast_to(amax, (1, 1, 128))


def kernel(input_logits, cu_seqlens_q, relative_idx, batch_offsets,
           cu_filtered, temperatures, num_transfer, top_k, top_p):
  del batch_offsets, num_transfer
  rows, v = input_logits.shape
  t_total = relative_idx.shape[0]
  nb = cu_filtered.shape[0] - 1
  width = v // 8

  counts = jnp.diff(cu_filtered)
  group_ids = jnp.repeat(jnp.arange(nb), counts, total_repeat_length=t_total)
  global_rows = (jnp.take(cu_seqlens_q[:-1], group_ids, axis=0)
                 + relative_idx).astype(jnp.int32)

  width_pad = ((width + 127) // 128) * 128
  logits3 = input_logits.reshape(rows, 8, width)
  logits3 = jnp.pad(logits3, ((0, 0), (0, 0), (0, width_pad - width)),
                    constant_values=_NEG)

  grid_spec = pltpu.PrefetchScalarGridSpec(
      num_scalar_prefetch=4,
      grid=(t_total,),
      in_specs=[
          pl.BlockSpec((1, 8, width_pad),
                       lambda t, rows_r, temps_r, ks_r, ps_r: (rows_r[t], 0, 0)),
      ],
      out_specs=[
          pl.BlockSpec((1, 1, 128),
                       lambda t, rows_r, temps_r, ks_r, ps_r: (t, 0, 0)),
          pl.BlockSpec((1, 1, 128),
                       lambda t, rows_r, temps_r, ks_r, ps_r: (t, 0, 0)),
      ],
  )

  score3, samp3 = pl.pallas_call(
      functools.partial(_sampler_kernel, width=width),
      grid_spec=grid_spec,
      out_shape=[
          jax.ShapeDtypeStruct((t_total, 1, 128), jnp.float32),
          jax.ShapeDtypeStruct((t_total, 1, 128), jnp.int32),
      ],
  )(global_rows, temperatures, top_k, top_p, logits3)

  return samp3[:, 0, 0], score3[:, 0, 0]
